# baseline (speedup 1.0000x reference)
import jax
import jax.numpy as jnp
from jax.experimental import pallas as pl

N_CLASSES = 1000
BATCH = 16384
B_BLK = 1024


def _onehot_body(idx_ref, out_ref):
    idx = idx_ref[0, 0, :].astype(jnp.int32)
    cls = jax.lax.broadcasted_iota(jnp.int32, (B_BLK, N_CLASSES), 1)
    out_ref[...] = (idx[:, None] == cls).astype(jnp.float32)


def kernel(inputs):
    idx3 = inputs.astype(jnp.int32).reshape(BATCH // B_BLK, 1, B_BLK)
    return pl.pallas_call(
        _onehot_body,
        grid=(BATCH // B_BLK,),
        in_specs=[pl.BlockSpec((1, 1, B_BLK), lambda i: (i, 0, 0))],
        out_specs=pl.BlockSpec((B_BLK, N_CLASSES), lambda i: (i, 0)),
        out_shape=jax.ShapeDtypeStruct((BATCH, N_CLASSES), jnp.float32),
    )(idx3)
